# Initial kernel scaffold; baseline (speedup 1.0000x reference)
import jax
import jax.numpy as jnp
from jax.experimental import pallas as pl


def kernel(unary, deltas, index1, index2):
    raise NotImplementedError("write your pallas kernel here")


# speedup vs baseline: 3.6771x; 3.6771x over previous
import functools

import jax
import jax.numpy as jnp
from jax import lax
from jax.experimental import pallas as pl
from jax.experimental.pallas import tpu as pltpu
from jax.experimental.pallas import tpu_sc as plsc

N = 131072
U = 64
NC = 2
NS = 16

EPT = N * U // NS
WCH = 16384


def _sc_body(idx1, idx2, h1, h2, rawbuf, flatbuf, onesbuf, zerobuf, sem):
  c = lax.axis_index("c")
  s = lax.axis_index("s")

  def _binit(i, carry):
    onesbuf[pl.ds(i * 16, 16)] = jnp.full((16,), 1.0, jnp.float32)
    zerobuf[pl.ds(i * 16, 16)] = jnp.zeros((16,), jnp.float32)
    return carry
  lax.fori_loop(0, WCH // 16, _binit, 0)

  def zero_phase(h_ref):
    for w in range(EPT // WCH):
      e0 = s * EPT + w * WCH
      pltpu.sync_copy(zerobuf, h_ref.at[pl.ds(e0, WCH)])

  @pl.when(c == 0)
  def _():
    zero_phase(h1)

  @pl.when(c == 1)
  def _():
    zero_phase(h2)

  plsc.subcore_barrier()

  jcs = [j0 + lax.iota(jnp.int32, 16) for j0 in range(0, U, 16)]

  def scatter_phase(idx_ref, h_ref):
    for w in range(EPT // WCH):
      e0 = s * EPT + w * WCH
      pltpu.sync_copy(idx_ref.at[pl.ds(e0, WCH)], rawbuf)

      def body(r, carry):
        for c4 in range(4):
          o = r * U + c4 * 16
          v = rawbuf[pl.ds(o, 16)]
          flatbuf[pl.ds(o, 16)] = lax.shift_left(v, 6) + jcs[c4]
        return carry
      lax.fori_loop(0, WCH // U, body, 0)
      pltpu.async_copy(onesbuf, h_ref.at[flatbuf], sem).wait()

  @pl.when(c == 0)
  def _():
    scatter_phase(idx1, h1)

  @pl.when(c == 1)
  def _():
    scatter_phase(idx2, h2)


_sc_masks = functools.partial(
    pl.kernel,
    out_type=(
        jax.ShapeDtypeStruct((N * U,), jnp.float32),
        jax.ShapeDtypeStruct((N * U,), jnp.float32),
    ),
    mesh=plsc.VectorSubcoreMesh(core_axis_name="c", subcore_axis_name="s"),
    scratch_types=[
        pltpu.VMEM((WCH,), jnp.int32),
        pltpu.VMEM((WCH,), jnp.int32),
        pltpu.VMEM((WCH,), jnp.float32),
        pltpu.VMEM((WCH,), jnp.float32),
        pltpu.SemaphoreType.DMA,
    ],
)(_sc_body)


def _combine_body(d_ref, h1_ref, h2_ref, o1_ref, ob_ref):
  d = d_ref[...]
  o1_ref[...] = (d[:, :U] * (1.0 - h1_ref[...])
                 + d[:, U:2 * U] * (1.0 - h2_ref[...]))
  ob_ref[...] = d[:, 2 * U:]


def _tc_combine(deltas, h1, h2):
  br = 2048
  spec = pl.BlockSpec((br, U), lambda i: (i, 0))
  return pl.pallas_call(
      _combine_body,
      out_shape=(jax.ShapeDtypeStruct((N, U), jnp.float32),
                 jax.ShapeDtypeStruct((N, U), jnp.float32)),
      grid=(N // br,),
      in_specs=[pl.BlockSpec((br, 3 * U), lambda i: (i, 0)), spec, spec],
      out_specs=(spec, spec),
  )(deltas, h1, h2)


def kernel(unary, deltas, index1, index2):
  h1, h2 = _sc_masks(index1.reshape(-1), index2.reshape(-1))
  return _tc_combine(deltas, h1.reshape(N, U), h2.reshape(N, U))


# speedup vs baseline: 88.8776x; 24.1703x over previous
import functools

import jax
import jax.numpy as jnp
from jax import lax
from jax.experimental import pallas as pl
from jax.experimental.pallas import tpu as pltpu
from jax.experimental.pallas import tpu_sc as plsc

N = 131072
U = 64
NC = 2
NS = 16

HALF = N // 2
COLS_PER_TILE = U // NS
WIN = 16384


def _sc_body(idxt1, idxt2, g1, g2, rawbuf, chunkbuf, sem):
  c = lax.axis_index("c")
  s = lax.axis_index("s")

  ones16 = jnp.full((16,), 1.0, jnp.float32)
  zeros16 = jnp.zeros((16,), jnp.float32)

  def build_masks(idxt, g):
    for cid in range(2 * COLS_PER_TILE):
      jcol = s * COLS_PER_TILE + (cid >> 1)
      h = cid & 1
      base = jcol * N

      @plsc.parallel_loop(0, HALF // 16, 1, unroll=8)
      def _init(i):
        chunkbuf[pl.ds(i * 16, 16)] = ones16

      for w in range(N // WIN):
        pltpu.sync_copy(idxt.at[pl.ds(base + w * WIN, WIN)], rawbuf)

        @plsc.parallel_loop(0, WIN // 16, 1, unroll=4)
        def _scan(i):
          v = rawbuf[pl.ds(i * 16, 16)]
          in_half = lax.shift_right_logical(v, 16) == h
          local = lax.bitwise_and(v, HALF - 1)
          plsc.store_scatter(chunkbuf, [local], zeros16, mask=in_half)

      pltpu.sync_copy(chunkbuf, g.at[pl.ds(base + h * HALF, HALF)])

  @pl.when(c == 0)
  def _():
    build_masks(idxt1, g1)

  @pl.when(c == 1)
  def _():
    build_masks(idxt2, g2)


_sc_masks = functools.partial(
    pl.kernel,
    out_type=(
        jax.ShapeDtypeStruct((U * N,), jnp.float32),
        jax.ShapeDtypeStruct((U * N,), jnp.float32),
    ),
    mesh=plsc.VectorSubcoreMesh(core_axis_name="c", subcore_axis_name="s"),
    compiler_params=pltpu.CompilerParams(needs_layout_passes=False),
    scratch_types=[
        pltpu.VMEM((WIN,), jnp.int32),
        pltpu.VMEM((HALF,), jnp.float32),
        pltpu.SemaphoreType.DMA,
    ],
)(_sc_body)


def _combine_body(d_ref, g1_ref, g2_ref, o1_ref, ob_ref):
  d = d_ref[...]
  o1_ref[...] = d[:, :U] * g1_ref[...] + d[:, U:2 * U] * g2_ref[...]
  ob_ref[...] = d[:, 2 * U:]


def _tc_combine(deltas, g1, g2):
  br = 2048
  spec = pl.BlockSpec((br, U), lambda i: (i, 0))
  return pl.pallas_call(
      _combine_body,
      out_shape=(jax.ShapeDtypeStruct((N, U), jnp.float32),
                 jax.ShapeDtypeStruct((N, U), jnp.float32)),
      grid=(N // br,),
      in_specs=[pl.BlockSpec((br, 3 * U), lambda i: (i, 0)), spec, spec],
      out_specs=(spec, spec),
  )(deltas, g1, g2)


def kernel(unary, deltas, index1, index2):
  g1f, g2f = _sc_masks(index1.T.reshape(-1), index2.T.reshape(-1))
  g1 = g1f.reshape(U, N).T
  g2 = g2f.reshape(U, N).T
  return _tc_combine(deltas, g1, g2)
